# baseline (speedup 1.0000x reference)
import functools

import jax
import jax.numpy as jnp
from jax import lax
from jax.experimental import pallas as pl
from jax.experimental.pallas import tpu as pltpu
from jax.experimental.pallas import tpu_sc as plsc

EMB = 128
OUT_EMB = 256
INT_EMB = 64
NSPH = 7
NRAD = 6
CUTOFF = 5.0
PEXP = 5

NE = 160000
NT = 320000
NCHUNK = 160
TPS = NCHUNK * 128
NTP = 16 * TPS
EHALF = NE // 2
NEPAD = 163840
NPASS = 16
RNG = 5120
ACCROWS = 5248
DUMP = 5120

ETILE = 2000


def _silu(x):
    return x * jax.lax.logistic(x)


def _wspec(shape):
    return pl.BlockSpec(shape, lambda i: (0,) * len(shape))


def _rowspec(cols):
    return pl.BlockSpec((ETILE, cols), lambda i: (i, 0))


SROWS = 200
SCAP = SROWS * 128

def _trip_agg_body(cgi2, csb2, cdst2, cst, ncht, e2d, sb, agg,
                   tb1, tb2, gi8, sb8, dst8, gbuf, sbb, zb, acc,
                   semG, semS):
    c = lax.axis_index("c")
    s = lax.axis_index("s")

    pltpu.sync_copy(cst, tb1)
    pltpu.sync_copy(ncht, tb2)

    def _zb(t, carry):
        zb[t // 8, pl.ds((t % 8) * 16, 16)] = jnp.zeros((16,), jnp.float32)
        return carry
    lax.fori_loop(0, 16 * 8, _zb, 0)

    for r in range(NPASS):
        lo = (c * 16 + r) * RNG
        prow = RNG // 16

        st = jnp.where(c == 0, tb1[s, pl.ds(0, 16)][r],
                       tb1[s, pl.ds(16, 16)][r])
        n = jnp.where(c == 0, tb2[s, pl.ds(0, 16)][r],
                      tb2[s, pl.ds(16, 16)][r])

        def _z(j, carry):
            pltpu.sync_copy(zb, acc.at[pl.ds(s * (RNG // 16) + j * 16, 16)])
            return carry
        lax.fori_loop(0, RNG // (16 * 16), _z, 0)
        plsc.subcore_barrier()

        def _proc(jj, carry):
            j = s * SROWS + st + jj
            jb = pl.multiple_of((j // 8) * 8, 8)
            o = j - jb
            pltpu.sync_copy(cgi2.at[pl.ds(jb, 8)], gi8)
            pltpu.sync_copy(csb2.at[pl.ds(jb, 8)], sb8)
            pltpu.sync_copy(cdst2.at[pl.ds(jb, 8)], dst8)
            pltpu.async_copy(e2d.at[gi8.at[o]], gbuf, semG)
            pltpu.async_copy(sb.at[sb8.at[o]], sbb, semS)
            pltpu.make_async_copy(e2d.at[gi8.at[o]], gbuf, semG).wait()
            pltpu.make_async_copy(sb.at[sb8.at[o]], sbb, semS).wait()

            def _mul(q, cc):
                rw = q // 4
                kk = (q % 4) * 16
                gbuf[rw, pl.ds(kk, 16)] = (gbuf[rw, pl.ds(kk, 16)]
                                           * sbb[rw, pl.ds(kk, 16)])
                return cc
            lax.fori_loop(0, 512, _mul, 0)
            pltpu.sync_copy(gbuf, acc.at[dst8.at[o]], add=True)
            return carry
        lax.fori_loop(0, n, _proc, 0)

        plsc.subcore_barrier()
        pltpu.sync_copy(acc.at[pl.ds(s * prow, prow)],
                        agg.at[pl.ds(lo + s * prow, prow)])
        plsc.subcore_barrier()


def _trip_agg(cgi2, csb2, cdst2, cst, ncht, e2d128, sb128):
    f = pl.kernel(
        _trip_agg_body,
        out_type=jax.ShapeDtypeStruct((NEPAD, 128), jnp.float32),
        mesh=plsc.VectorSubcoreMesh(core_axis_name="c", subcore_axis_name="s"),
        scratch_types=[
            pltpu.VMEM((16, 32), jnp.int32),
            pltpu.VMEM((16, 32), jnp.int32),
            pltpu.VMEM((8, 128), jnp.int32),
            pltpu.VMEM((8, 128), jnp.int32),
            pltpu.VMEM((8, 128), jnp.int32),
            pltpu.VMEM((128, 128), jnp.float32),
            pltpu.VMEM((128, 128), jnp.float32),
            pltpu.VMEM((16, 128), jnp.float32),
            pltpu.VMEM_SHARED((ACCROWS, 128), jnp.float32),
            pltpu.SemaphoreType.DMA,
            pltpu.SemaphoreType.DMA,
        ],
    )
    return f(cgi2, csb2, cdst2, cst, ncht, e2d128, sb128)


def _prep_triplet_buckets(id_reduce, id_expand):
    idr = jnp.full((NTP,), 2_000_000, jnp.int32).at[:NT].set(
        id_reduce.astype(jnp.int32))
    gix = jnp.zeros((NTP,), jnp.int32).at[:NT].set(id_expand.astype(jnp.int32))
    bt = jnp.minimum(idr // RNG, 32)
    keys = bt.reshape(16, TPS)
    oh = (keys[:, :, None] == jnp.arange(33)[None, None, :]).astype(jnp.int32)
    ranks = jnp.cumsum(oh, axis=1)
    rank_t = jnp.take_along_axis(ranks, keys[:, :, None], 2)[:, :, 0] - 1
    cnt = ranks[:, -1, :]
    cap = ((cnt + 127) // 128) * 128
    astart = jnp.concatenate(
        [jnp.zeros((16, 1), jnp.int32), jnp.cumsum(cap, axis=1)[:, :32]], 1)
    sidx = jnp.arange(16, dtype=jnp.int32)[:, None]
    flatpos = (sidx * SCAP + jnp.take_along_axis(astart, keys, 1)
               + rank_t).reshape(-1)
    cdst_val = jnp.where(bt < 32, idr - bt * RNG, DUMP)
    cgi2 = jnp.zeros((16 * SCAP,), jnp.int32).at[flatpos].set(gix)
    csb2 = jnp.full((16 * SCAP,), NT, jnp.int32).at[flatpos].set(
        jnp.arange(NTP, dtype=jnp.int32))
    cdst2 = jnp.full((16 * SCAP,), DUMP, jnp.int32).at[flatpos].set(cdst_val)
    cst = (astart[:, :32] // 128).astype(jnp.int32)
    ncht = (cap[:, :32] // 128).astype(jnp.int32)
    return (cgi2.reshape(-1, 128), csb2.reshape(-1, 128),
            cdst2.reshape(-1, 128), cst, ncht)


def _blockA_body(x_ref, rbf_ref, Wji_ref, bji_ref, Wkj_ref, bkj_ref, RB_ref,
                 down_ref, e1_ref, d_ref):
    x = x_ref[...]
    e1_ref[...] = _silu(x @ Wji_ref[...] + bji_ref[...])
    e2 = _silu(x @ Wkj_ref[...] + bkj_ref[...]) * (rbf_ref[...] @ RB_ref[...])
    e2d = _silu(e2 @ down_ref[...])
    d_ref[...] = jnp.concatenate(
        [e2d, jnp.zeros((ETILE, 64), jnp.float32)], axis=1)


def _blockA(x, rbf8, b):
    n = x.shape[0]
    RB = jnp.pad(b['rbf1'] @ b['rbf2'], ((0, 2), (0, 0)))
    return pl.pallas_call(
        _blockA_body,
        grid=(n // ETILE,),
        in_specs=[_rowspec(EMB), _rowspec(8),
                  _wspec((EMB, EMB)), _wspec((1, EMB)),
                  _wspec((EMB, EMB)), _wspec((1, EMB)),
                  _wspec((8, EMB)), _wspec((EMB, INT_EMB))],
        out_specs=[_rowspec(EMB), _rowspec(128)],
        out_shape=[jax.ShapeDtypeStruct((n, EMB), jnp.float32),
                   jax.ShapeDtypeStruct((n, 128), jnp.float32)],
    )(x, rbf8, b['Wji'], b['bji'].reshape(1, -1), b['Wkj'],
      b['bkj'].reshape(1, -1), RB, b['down'])


def _blockB_body(agg_ref, e1_ref, x_ref, rbf_ref, up_ref,
                 bW1, bb1, bW2, bb2, Wfin, bfin,
                 aW1, ab1, aW2, ab2, aW3, ab3, aW4, ab4,
                 Wrbf, xn_ref, gx_ref):
    u = _silu(agg_ref[:, 0:64] @ up_ref[...])
    hm = e1_ref[...] + u
    hm = hm + _silu(_silu(hm @ bW1[...] + bb1[...]) @ bW2[...] + bb2[...])
    hn = _silu(hm @ Wfin[...] + bfin[...]) + x_ref[...]
    hn = hn + _silu(_silu(hn @ aW1[...] + ab1[...]) @ aW2[...] + ab2[...])
    hn = hn + _silu(_silu(hn @ aW3[...] + ab3[...]) @ aW4[...] + ab4[...])
    xn_ref[...] = hn
    gx_ref[...] = (rbf_ref[...] @ Wrbf[...]) * hn


def _blockB(agg, e1, x, rbf8, b, o):
    n = x.shape[0]
    (bW1, bb1, bW2, bb2), = b['before']
    (aW1, ab1, aW2, ab2), (aW3, ab3, aW4, ab4) = b['after']
    Wrbf8 = jnp.pad(o['Wrbf'], ((0, 2), (0, 0)))
    r = lambda v: v.reshape(1, -1)
    return pl.pallas_call(
        _blockB_body,
        grid=(n // ETILE,),
        in_specs=[_rowspec(128), _rowspec(EMB), _rowspec(EMB), _rowspec(8),
                  _wspec((INT_EMB, EMB)),
                  _wspec((EMB, EMB)), _wspec((1, EMB)),
                  _wspec((EMB, EMB)), _wspec((1, EMB)),
                  _wspec((EMB, EMB)), _wspec((1, EMB)),
                  _wspec((EMB, EMB)), _wspec((1, EMB)),
                  _wspec((EMB, EMB)), _wspec((1, EMB)),
                  _wspec((EMB, EMB)), _wspec((1, EMB)),
                  _wspec((EMB, EMB)), _wspec((1, EMB)),
                  _wspec((8, EMB))],
        out_specs=[_rowspec(EMB), _rowspec(EMB)],
        out_shape=[jax.ShapeDtypeStruct((n, EMB), jnp.float32),
                   jax.ShapeDtypeStruct((n, EMB), jnp.float32)],
    )(agg, e1, x, rbf8, b['up'],
      bW1, r(bb1), bW2, r(bb2), b['Wfin'], r(b['bfin']),
      aW1, r(ab1), aW2, r(ab2), aW3, r(ab3), aW4, r(ab4), Wrbf8)


def _envelope(x):
    p = PEXP
    a = -(p + 1) * (p + 2) / 2.0
    b = p * (p + 2.0)
    c = -p * (p + 1) / 2.0
    xs = jnp.clip(x, 1e-9, None)
    env = 1.0 / xs + a * xs ** (p - 1) + b * xs ** p + c * xs ** (p + 1)
    return jnp.where(x < 1.0, env, 0.0)


def _radial(x):
    freqs = jnp.pi * jnp.arange(1, NRAD + 1, dtype=jnp.float32)
    return _envelope(x)[:, None] * jnp.sin(freqs[None, :] * x[:, None])


def kernel(Z, R, batch_seg, idnb_i, idnb_j, id_expand_kj, id_reduce_ji,
           id3dnb_i, id3dnb_j, id3dnb_k, params):
    n_atoms = Z.shape[0]
    n_graph = 512

    Ri = R[idnb_i]
    Rj = R[idnb_j]
    Dij = jnp.sqrt(jnp.maximum(jnp.sum((Ri - Rj) ** 2, -1), 1e-12))
    rbf = _radial(Dij / CUTOFF)
    rbf8 = jnp.pad(rbf, ((0, 0), (0, 2)))

    R1 = R[id3dnb_j] - R[id3dnb_i]
    R2 = R[id3dnb_k] - R[id3dnb_j]
    xdot = jnp.sum(R1 * R2, -1)
    ycr = jnp.sqrt(jnp.sum(jnp.cross(R1, R2) ** 2, -1) + 1e-9)
    angles = jnp.arctan2(ycr, xdot)
    rad_t = _radial((Dij / CUTOFF)[id_expand_kj])
    ls = jnp.arange(NSPH, dtype=jnp.float32)
    angular = jnp.cos(ls[None, :] * angles[:, None])
    sbf = (angular[:, :, None] * rad_t[:, None, :]).reshape(-1, NSPH * NRAD)

    cgi2, csb2, cdst2, cst, ncht = _prep_triplet_buckets(
        id_reduce_ji, id_expand_kj)

    h = params['z_emb'][Z]
    rbf_e = _silu(rbf @ params['emb_rbf_W'] + params['emb_rbf_b'])
    x = _silu(jnp.concatenate([h[idnb_i], h[idnb_j], rbf_e], -1)
              @ params['emb_cat_W'] + params['emb_cat_b'])

    def atom_chain(o, t):
        t = t @ o['Wup']
        for (W, b) in o['dense']:
            t = _silu(t @ W + b)
        return t @ o['Wout']

    o0 = params['out'][0]
    gx = (rbf @ o0['Wrbf']) * x
    t0 = jax.ops.segment_sum(gx, idnb_i, num_segments=n_atoms)
    P_atom = atom_chain(o0, t0)

    for i in range(3):
        b = params['int'][i]
        o = params['out'][i + 1]
        e1, e2d128 = _blockA(x, rbf8, b)
        SB = b['sbf1'] @ b['sbf2']
        sb128 = jnp.zeros((NTP, 128), jnp.float32)
        sb128 = sb128.at[:NT, 0:64].set(sbf @ SB)
        agg = _trip_agg(cgi2, csb2, cdst2, cst, ncht, e2d128, sb128)[:NE]
        x, gx = _blockB(agg, e1, x, rbf8, b, o)
        t = jax.ops.segment_sum(gx, idnb_i, num_segments=n_atoms)
        P_atom = P_atom + atom_chain(o, t)

    energy = jax.ops.segment_sum(P_atom, batch_seg, num_segments=n_graph)
    return energy


# speedup vs baseline: 1.0151x; 1.0151x over previous
import functools

import jax
import jax.numpy as jnp
from jax import lax
from jax.experimental import pallas as pl
from jax.experimental.pallas import tpu as pltpu
from jax.experimental.pallas import tpu_sc as plsc

EMB = 128
OUT_EMB = 256
INT_EMB = 64
NSPH = 7
NRAD = 6
CUTOFF = 5.0
PEXP = 5

NE = 160000
NT = 320000
NCHUNK = 160
TPS = NCHUNK * 128
NTP = 16 * TPS
EHALF = NE // 2
NEPAD = 163840
NPASS = 16
RNG = 5120
ACCROWS = 5248
DUMP = 5120

ETILE = 2000


def _silu(x):
    return x * jax.lax.logistic(x)


def _wspec(shape):
    return pl.BlockSpec(shape, lambda i: (0,) * len(shape))


def _rowspec(cols):
    return pl.BlockSpec((ETILE, cols), lambda i: (i, 0))


SROWS = 200
SCAP = SROWS * 128

def _trip_agg_body(cgi2, csb2, cdst2, cst, ncht, e2d, sb, agg,
                   tb1, tb2, gi8, sb8, dst8, gbuf, sbb,
                   gi8b, sb8b, dst8b, gbufb, sbbb, zb, acc,
                   semG, semS, semGb, semSb):
    c = lax.axis_index("c")
    s = lax.axis_index("s")

    pltpu.sync_copy(cst, tb1)
    pltpu.sync_copy(ncht, tb2)

    def _zb(t, carry):
        zb[t // 8, pl.ds((t % 8) * 16, 16)] = jnp.zeros((16,), jnp.float32)
        return carry
    lax.fori_loop(0, 16 * 8, _zb, 0)

    for r in range(NPASS):
        lo = (c * 16 + r) * RNG
        prow = RNG // 16

        st = jnp.where(c == 0, tb1[s, pl.ds(0, 16)][r],
                       tb1[s, pl.ds(16, 16)][r])
        n = jnp.where(c == 0, tb2[s, pl.ds(0, 16)][r],
                      tb2[s, pl.ds(16, 16)][r])

        def _z(j, carry):
            pltpu.sync_copy(zb, acc.at[pl.ds(s * (RNG // 16) + j * 16, 16)])
            return carry
        lax.fori_loop(0, RNG // (16 * 16), _z, 0)
        plsc.subcore_barrier()

        gstart = s * SROWS + st
        sets = ((gi8, sb8, dst8, gbuf, sbb, semG, semS),
                (gi8b, sb8b, dst8b, gbufb, sbbb, semGb, semSb))

        def _fire(j, gi8x, sb8x, dst8x, gbufx, sbbx, semGx, semSx):
            jb = pl.multiple_of((j // 8) * 8, 8)
            o = j - jb
            pltpu.sync_copy(cgi2.at[pl.ds(jb, 8)], gi8x)
            pltpu.sync_copy(csb2.at[pl.ds(jb, 8)], sb8x)
            pltpu.sync_copy(cdst2.at[pl.ds(jb, 8)], dst8x)
            pltpu.async_copy(e2d.at[gi8x.at[o]], gbufx, semGx)
            pltpu.async_copy(sb.at[sb8x.at[o]], sbbx, semSx)

        def _finish(j, gi8x, sb8x, dst8x, gbufx, sbbx, semGx, semSx):
            jb = pl.multiple_of((j // 8) * 8, 8)
            o = j - jb
            pltpu.make_async_copy(e2d.at[gi8x.at[o]], gbufx, semGx).wait()
            pltpu.make_async_copy(sb.at[sb8x.at[o]], sbbx, semSx).wait()

            def _mul(q, cc):
                rw = q // 4
                kk = (q % 4) * 16
                gbufx[rw, pl.ds(kk, 16)] = (gbufx[rw, pl.ds(kk, 16)]
                                            * sbbx[rw, pl.ds(kk, 16)])
                return cc
            lax.fori_loop(0, 512, _mul, 0)
            pltpu.sync_copy(gbufx, acc.at[dst8x.at[o]], add=True)

        @pl.when(n > 0)
        def _():
            _fire(gstart, *sets[0])

        def _proc(k, carry):
            @pl.when((k % 2 == 0) & (k + 1 < n))
            def _():
                _fire(gstart + k + 1, *sets[1])

            @pl.when((k % 2 == 1) & (k + 1 < n))
            def _():
                _fire(gstart + k + 1, *sets[0])

            @pl.when(k % 2 == 0)
            def _():
                _finish(gstart + k, *sets[0])

            @pl.when(k % 2 == 1)
            def _():
                _finish(gstart + k, *sets[1])
            return carry
        lax.fori_loop(0, n, _proc, 0)

        plsc.subcore_barrier()
        pltpu.sync_copy(acc.at[pl.ds(s * prow, prow)],
                        agg.at[pl.ds(lo + s * prow, prow)])
        plsc.subcore_barrier()


def _trip_agg(cgi2, csb2, cdst2, cst, ncht, e2d128, sb128):
    f = pl.kernel(
        _trip_agg_body,
        out_type=jax.ShapeDtypeStruct((NEPAD, 128), jnp.float32),
        mesh=plsc.VectorSubcoreMesh(core_axis_name="c", subcore_axis_name="s"),
        scratch_types=[
            pltpu.VMEM((16, 32), jnp.int32),
            pltpu.VMEM((16, 32), jnp.int32),
            pltpu.VMEM((8, 128), jnp.int32),
            pltpu.VMEM((8, 128), jnp.int32),
            pltpu.VMEM((8, 128), jnp.int32),
            pltpu.VMEM((128, 128), jnp.float32),
            pltpu.VMEM((128, 128), jnp.float32),
            pltpu.VMEM((8, 128), jnp.int32),
            pltpu.VMEM((8, 128), jnp.int32),
            pltpu.VMEM((8, 128), jnp.int32),
            pltpu.VMEM((128, 128), jnp.float32),
            pltpu.VMEM((128, 128), jnp.float32),
            pltpu.VMEM((16, 128), jnp.float32),
            pltpu.VMEM_SHARED((ACCROWS, 128), jnp.float32),
            pltpu.SemaphoreType.DMA,
            pltpu.SemaphoreType.DMA,
            pltpu.SemaphoreType.DMA,
            pltpu.SemaphoreType.DMA,
        ],
    )
    return f(cgi2, csb2, cdst2, cst, ncht, e2d128, sb128)


def _prep_triplet_buckets(id_reduce, id_expand):
    idr = jnp.full((NTP,), 2_000_000, jnp.int32).at[:NT].set(
        id_reduce.astype(jnp.int32))
    gix = jnp.zeros((NTP,), jnp.int32).at[:NT].set(id_expand.astype(jnp.int32))
    bt = jnp.minimum(idr // RNG, 32)
    keys = bt.reshape(16, TPS)
    oh = (keys[:, :, None] == jnp.arange(33)[None, None, :]).astype(jnp.int32)
    ranks = jnp.cumsum(oh, axis=1)
    rank_t = jnp.take_along_axis(ranks, keys[:, :, None], 2)[:, :, 0] - 1
    cnt = ranks[:, -1, :]
    cap = ((cnt + 127) // 128) * 128
    astart = jnp.concatenate(
        [jnp.zeros((16, 1), jnp.int32), jnp.cumsum(cap, axis=1)[:, :32]], 1)
    sidx = jnp.arange(16, dtype=jnp.int32)[:, None]
    flatpos = (sidx * SCAP + jnp.take_along_axis(astart, keys, 1)
               + rank_t).reshape(-1)
    cdst_val = jnp.where(bt < 32, idr - bt * RNG, DUMP)
    cgi2 = jnp.zeros((16 * SCAP,), jnp.int32).at[flatpos].set(gix)
    csb2 = jnp.full((16 * SCAP,), NT, jnp.int32).at[flatpos].set(
        jnp.arange(NTP, dtype=jnp.int32))
    cdst2 = jnp.full((16 * SCAP,), DUMP, jnp.int32).at[flatpos].set(cdst_val)
    cst = (astart[:, :32] // 128).astype(jnp.int32)
    ncht = (cap[:, :32] // 128).astype(jnp.int32)
    return (cgi2.reshape(-1, 128), csb2.reshape(-1, 128),
            cdst2.reshape(-1, 128), cst, ncht)


def _blockA_body(x_ref, rbf_ref, Wji_ref, bji_ref, Wkj_ref, bkj_ref, RB_ref,
                 down_ref, e1_ref, d_ref):
    x = x_ref[...]
    e1_ref[...] = _silu(x @ Wji_ref[...] + bji_ref[...])
    e2 = _silu(x @ Wkj_ref[...] + bkj_ref[...]) * (rbf_ref[...] @ RB_ref[...])
    e2d = _silu(e2 @ down_ref[...])
    d_ref[...] = jnp.concatenate(
        [e2d, jnp.zeros((ETILE, 64), jnp.float32)], axis=1)


def _blockA(x, rbf8, b):
    n = x.shape[0]
    RB = jnp.pad(b['rbf1'] @ b['rbf2'], ((0, 2), (0, 0)))
    return pl.pallas_call(
        _blockA_body,
        grid=(n // ETILE,),
        in_specs=[_rowspec(EMB), _rowspec(8),
                  _wspec((EMB, EMB)), _wspec((1, EMB)),
                  _wspec((EMB, EMB)), _wspec((1, EMB)),
                  _wspec((8, EMB)), _wspec((EMB, INT_EMB))],
        out_specs=[_rowspec(EMB), _rowspec(128)],
        out_shape=[jax.ShapeDtypeStruct((n, EMB), jnp.float32),
                   jax.ShapeDtypeStruct((n, 128), jnp.float32)],
    )(x, rbf8, b['Wji'], b['bji'].reshape(1, -1), b['Wkj'],
      b['bkj'].reshape(1, -1), RB, b['down'])


def _blockB_body(agg_ref, e1_ref, x_ref, rbf_ref, up_ref,
                 bW1, bb1, bW2, bb2, Wfin, bfin,
                 aW1, ab1, aW2, ab2, aW3, ab3, aW4, ab4,
                 Wrbf, xn_ref, gx_ref):
    u = _silu(agg_ref[:, 0:64] @ up_ref[...])
    hm = e1_ref[...] + u
    hm = hm + _silu(_silu(hm @ bW1[...] + bb1[...]) @ bW2[...] + bb2[...])
    hn = _silu(hm @ Wfin[...] + bfin[...]) + x_ref[...]
    hn = hn + _silu(_silu(hn @ aW1[...] + ab1[...]) @ aW2[...] + ab2[...])
    hn = hn + _silu(_silu(hn @ aW3[...] + ab3[...]) @ aW4[...] + ab4[...])
    xn_ref[...] = hn
    gx_ref[...] = (rbf_ref[...] @ Wrbf[...]) * hn


def _blockB(agg, e1, x, rbf8, b, o):
    n = x.shape[0]
    (bW1, bb1, bW2, bb2), = b['before']
    (aW1, ab1, aW2, ab2), (aW3, ab3, aW4, ab4) = b['after']
    Wrbf8 = jnp.pad(o['Wrbf'], ((0, 2), (0, 0)))
    r = lambda v: v.reshape(1, -1)
    return pl.pallas_call(
        _blockB_body,
        grid=(n // ETILE,),
        in_specs=[_rowspec(128), _rowspec(EMB), _rowspec(EMB), _rowspec(8),
                  _wspec((INT_EMB, EMB)),
                  _wspec((EMB, EMB)), _wspec((1, EMB)),
                  _wspec((EMB, EMB)), _wspec((1, EMB)),
                  _wspec((EMB, EMB)), _wspec((1, EMB)),
                  _wspec((EMB, EMB)), _wspec((1, EMB)),
                  _wspec((EMB, EMB)), _wspec((1, EMB)),
                  _wspec((EMB, EMB)), _wspec((1, EMB)),
                  _wspec((EMB, EMB)), _wspec((1, EMB)),
                  _wspec((8, EMB))],
        out_specs=[_rowspec(EMB), _rowspec(EMB)],
        out_shape=[jax.ShapeDtypeStruct((n, EMB), jnp.float32),
                   jax.ShapeDtypeStruct((n, EMB), jnp.float32)],
    )(agg, e1, x, rbf8, b['up'],
      bW1, r(bb1), bW2, r(bb2), b['Wfin'], r(b['bfin']),
      aW1, r(ab1), aW2, r(ab2), aW3, r(ab3), aW4, r(ab4), Wrbf8)


def _envelope(x):
    p = PEXP
    a = -(p + 1) * (p + 2) / 2.0
    b = p * (p + 2.0)
    c = -p * (p + 1) / 2.0
    xs = jnp.clip(x, 1e-9, None)
    env = 1.0 / xs + a * xs ** (p - 1) + b * xs ** p + c * xs ** (p + 1)
    return jnp.where(x < 1.0, env, 0.0)


def _radial(x):
    freqs = jnp.pi * jnp.arange(1, NRAD + 1, dtype=jnp.float32)
    return _envelope(x)[:, None] * jnp.sin(freqs[None, :] * x[:, None])


def kernel(Z, R, batch_seg, idnb_i, idnb_j, id_expand_kj, id_reduce_ji,
           id3dnb_i, id3dnb_j, id3dnb_k, params):
    n_atoms = Z.shape[0]
    n_graph = 512

    Ri = R[idnb_i]
    Rj = R[idnb_j]
    Dij = jnp.sqrt(jnp.maximum(jnp.sum((Ri - Rj) ** 2, -1), 1e-12))
    rbf = _radial(Dij / CUTOFF)
    rbf8 = jnp.pad(rbf, ((0, 0), (0, 2)))

    R1 = R[id3dnb_j] - R[id3dnb_i]
    R2 = R[id3dnb_k] - R[id3dnb_j]
    xdot = jnp.sum(R1 * R2, -1)
    ycr = jnp.sqrt(jnp.sum(jnp.cross(R1, R2) ** 2, -1) + 1e-9)
    angles = jnp.arctan2(ycr, xdot)
    rad_t = _radial((Dij / CUTOFF)[id_expand_kj])
    ls = jnp.arange(NSPH, dtype=jnp.float32)
    angular = jnp.cos(ls[None, :] * angles[:, None])
    sbf = (angular[:, :, None] * rad_t[:, None, :]).reshape(-1, NSPH * NRAD)

    cgi2, csb2, cdst2, cst, ncht = _prep_triplet_buckets(
        id_reduce_ji, id_expand_kj)

    h = params['z_emb'][Z]
    rbf_e = _silu(rbf @ params['emb_rbf_W'] + params['emb_rbf_b'])
    x = _silu(jnp.concatenate([h[idnb_i], h[idnb_j], rbf_e], -1)
              @ params['emb_cat_W'] + params['emb_cat_b'])

    def atom_chain(o, t):
        t = t @ o['Wup']
        for (W, b) in o['dense']:
            t = _silu(t @ W + b)
        return t @ o['Wout']

    o0 = params['out'][0]
    gx = (rbf @ o0['Wrbf']) * x
    t0 = jax.ops.segment_sum(gx, idnb_i, num_segments=n_atoms)
    P_atom = atom_chain(o0, t0)

    for i in range(3):
        b = params['int'][i]
        o = params['out'][i + 1]
        e1, e2d128 = _blockA(x, rbf8, b)
        SB = b['sbf1'] @ b['sbf2']
        sb128 = jnp.zeros((NTP, 128), jnp.float32)
        sb128 = sb128.at[:NT, 0:64].set(sbf @ SB)
        agg = _trip_agg(cgi2, csb2, cdst2, cst, ncht, e2d128, sb128)[:NE]
        x, gx = _blockB(agg, e1, x, rbf8, b, o)
        t = jax.ops.segment_sum(gx, idnb_i, num_segments=n_atoms)
        P_atom = P_atom + atom_chain(o, t)

    energy = jax.ops.segment_sum(P_atom, batch_seg, num_segments=n_graph)
    return energy
